# baseline (speedup 1.0000x reference)
import jax
import jax.numpy as jnp
from jax import lax
from jax.experimental import pallas as pl
from jax.experimental.pallas import tpu as pltpu
from jax.experimental.pallas import tpu_sc as plsc

N = 10000
NP = 10240
D = 128
DE = 144
NG = 64
NCLS = 8
NC, NS, L = 2, 16, 16
NW = NC * NS
E_REAL = 320000 + N
EB = 64
IG = 27
G0 = 7
G1 = 5
NB0 = G0 * IG
NB1 = G1 * IG
E0 = NS * NB0 * EB
E_PAD = NS * (NB0 + NB1) * EB
RPT = NP // NS
BLK = 512
NBLK = NP // BLK


def _emit_outputs(h, ats_ref, atd_ref, hext_ref, asd_ref):
    hext_ref[:, :D] = h
    asrc = jnp.sum(h * ats_ref[...], axis=1, keepdims=True)
    adst = jnp.sum(h * atd_ref[...], axis=1, keepdims=True)
    col = lax.broadcasted_iota(jnp.int32, (h.shape[0], DE - D), 1)
    tail = jnp.where(col == 0, 1.0, jnp.where(col == 1, asrc, 0.0))
    hext_ref[:, D:] = tail
    col8 = lax.broadcasted_iota(jnp.int32, (h.shape[0], 8), 1)
    asd_ref[...] = jnp.where(col8 == 0, adst, 0.0)


def _dense_first_body(x_ref, w_ref, ats_ref, atd_ref, hext_ref, asd_ref):
    h = jnp.dot(x_ref[...], w_ref[...], preferred_element_type=jnp.float32)
    _emit_outputs(h, ats_ref, atd_ref, hext_ref, asd_ref)


def _node_activation(a_ref, b_ref, bias_ref):
    den = a_ref[:, D:D + 1] + b_ref[:, D:D + 1]
    num = a_ref[:, :D] + b_ref[:, :D]
    return jnp.maximum(num / jnp.maximum(den, 1e-16) + bias_ref[...], 0.0)


def _dense_mid_body(a_ref, b_ref, bias_ref, w_ref, ats_ref, atd_ref,
                    hext_ref, asd_ref):
    xv = _node_activation(a_ref, b_ref, bias_ref)
    h = jnp.dot(xv, w_ref[...], preferred_element_type=jnp.float32)
    _emit_outputs(h, ats_ref, atd_ref, hext_ref, asd_ref)


def _dense_outs():
    return (
        [pl.BlockSpec((BLK, DE), lambda i: (i, 0)),
         pl.BlockSpec((BLK, 8), lambda i: (i, 0))],
        [jax.ShapeDtypeStruct((NP, DE), jnp.float32),
         jax.ShapeDtypeStruct((NP, 8), jnp.float32)],
    )


def _dense_first(x_pad, W, ats, atd):
    out_specs, out_shape = _dense_outs()
    return pl.pallas_call(
        _dense_first_body,
        grid=(NBLK,),
        in_specs=[pl.BlockSpec((BLK, D), lambda i: (i, 0)),
                  pl.BlockSpec((D, D), lambda i: (0, 0)),
                  pl.BlockSpec((1, D), lambda i: (0, 0)),
                  pl.BlockSpec((1, D), lambda i: (0, 0))],
        out_specs=out_specs,
        out_shape=out_shape,
    )(x_pad, W, ats, atd)


def _dense_mid(accA, accB, bias, W, ats, atd):
    out_specs, out_shape = _dense_outs()
    return pl.pallas_call(
        _dense_mid_body,
        grid=(NBLK,),
        in_specs=[pl.BlockSpec((BLK, DE), lambda i: (i, 0)),
                  pl.BlockSpec((BLK, DE), lambda i: (i, 0)),
                  pl.BlockSpec((1, D), lambda i: (0, 0)),
                  pl.BlockSpec((D, D), lambda i: (0, 0)),
                  pl.BlockSpec((1, D), lambda i: (0, 0)),
                  pl.BlockSpec((1, D), lambda i: (0, 0))],
        out_specs=out_specs,
        out_shape=out_shape,
    )(accA, accB, bias, W, ats, atd)


def _final_body(a_ref, b_ref, bias_ref, s_ref, fcw_ref, fcb_ref, out_ref,
                pooled, cnt):
    i = pl.program_id(0)

    @pl.when(i == 0)
    def _():
        pooled[...] = jnp.zeros_like(pooled)
        cnt[...] = jnp.zeros_like(cnt)

    xv = _node_activation(a_ref, b_ref, bias_ref)
    sb = s_ref[...]
    pooled[...] += lax.dot_general(sb, xv, (((0,), (0,)), ((), ())),
                                   preferred_element_type=jnp.float32)
    cnt[...] += jnp.sum(sb, axis=0, keepdims=True)

    @pl.when(i == NBLK - 1)
    def _():
        c = jnp.maximum(cnt[...].reshape(NG, 1), 1.0)
        logits = jnp.dot(pooled[...] / c, fcw_ref[...],
                         preferred_element_type=jnp.float32) + fcb_ref[...]
        m = jnp.max(logits, axis=1, keepdims=True)
        e = jnp.exp(logits - m)
        out_ref[...] = e / jnp.sum(e, axis=1, keepdims=True)


def _final(accA, accB, bias, S, fcw, fcb):
    return pl.pallas_call(
        _final_body,
        grid=(NBLK,),
        in_specs=[pl.BlockSpec((BLK, DE), lambda i: (i, 0)),
                  pl.BlockSpec((BLK, DE), lambda i: (i, 0)),
                  pl.BlockSpec((1, D), lambda i: (0, 0)),
                  pl.BlockSpec((BLK, NG), lambda i: (i, 0)),
                  pl.BlockSpec((D, NCLS), lambda i: (0, 0)),
                  pl.BlockSpec((1, NCLS), lambda i: (0, 0))],
        out_specs=pl.BlockSpec((NG, NCLS), lambda i: (0, 0)),
        out_shape=jax.ShapeDtypeStruct((NG, NCLS), jnp.float32),
        scratch_shapes=[pltpu.VMEM((NG, D), jnp.float32),
                        pltpu.VMEM((1, NG), jnp.float32)],
    )(accA, accB, bias, S, fcw, fcb)


def _edge_body(hext_hbm, adst_hbm, srcA_hbm, dstA_hbm, srcB_hbm, dstB_hbm,
               out_hbm, adst_v, srcg, dstg, gbuf, wbuf, acc,
               sem_i, sem_g, sem_s):
    c = lax.axis_index("c")
    s = lax.axis_index("s")

    pltpu.async_copy(adst_hbm, adst_v, sem_i)
    zeros16 = jnp.zeros((L,), jnp.float32)

    def _zrow(r, carry):
        for v in range(DE // L):
            gbuf[0, r, pl.ds(v * L, L)] = zeros16
        return carry

    lax.fori_loop(0, EB, _zrow, 0)
    base = s * RPT

    def _zcopy(j, carry):
        pltpu.async_copy(gbuf.at[0], acc.at[pl.ds(base + j * EB, EB)], sem_s)
        return carry

    lax.fori_loop(0, RPT // EB, _zcopy, 0)
    pltpu.make_async_copy(adst_hbm, adst_v, sem_i).wait()

    def _zwait(j, carry):
        pltpu.make_async_copy(gbuf.at[0], acc.at[pl.ds(base + j * EB, EB)],
                              sem_s).wait()
        return carry

    lax.fori_loop(0, RPT // EB, _zwait, 0)
    plsc.subcore_barrier()

    def _run(nb, ngrp, src_hbm, dst_hbm):
        def _igrp(k):
            return (k // IG) % 2, k % IG

        def _idx_start(k):
            isl = (k // IG) % 2
            pltpu.async_copy(src_hbm.at[s, k // IG], srcg.at[isl], sem_i)
            pltpu.async_copy(dst_hbm.at[s, k // IG], dstg.at[isl], sem_i)

        def _idx_wait(k):
            isl = (k // IG) % 2
            pltpu.make_async_copy(src_hbm.at[s, k // IG], srcg.at[isl],
                                  sem_i).wait()
            pltpu.make_async_copy(dst_hbm.at[s, k // IG], dstg.at[isl],
                                  sem_i).wait()

        def _g_start(blk, sl):
            isl, bb = _igrp(blk)
            pltpu.async_copy(hext_hbm.at[srcg.at[isl, bb]], gbuf.at[sl],
                             sem_g)

        def _g_wait(blk, sl):
            isl, bb = _igrp(blk)
            pltpu.make_async_copy(hext_hbm.at[srcg.at[isl, bb]], gbuf.at[sl],
                                  sem_g).wait()

        def _s_start(blk, sl):
            isl, bb = _igrp(blk)
            pltpu.async_copy(gbuf.at[sl], acc.at[dstg.at[isl, bb]], sem_s,
                             add=True)

        def _s_wait(blk, sl):
            isl, bb = _igrp(blk)
            pltpu.make_async_copy(gbuf.at[sl], acc.at[dstg.at[isl, bb]],
                                  sem_s).wait()

        def _compute(blk, sl):
            isl, bb = _igrp(blk)
            for g in range(EB // L):
                rows = lax.iota(jnp.int32, L) + g * L
                cols = jnp.full((L,), D + 1, jnp.int32)
                asv = plsc.load_gather(gbuf.at[sl], [rows, cols])
                di = dstg[isl, bb, pl.ds(g * L, L)]
                adv = plsc.load_gather(adst_v, [di])
                al = asv + adv
                al = jnp.maximum(al, 0.0) + 0.2 * jnp.minimum(al, 0.0)
                wbuf[pl.ds(g * L, L)] = jnp.exp(jnp.minimum(al, 60.0))

            def _row(i, rc):
                for u in range(8):
                    r = 8 * i + u
                    wspl = plsc.load_gather(wbuf,
                                            [jnp.full((L,), r, jnp.int32)])
                    for v in range(DE // L):
                        gbuf[sl, r, pl.ds(v * L, L)] = (
                            gbuf[sl, r, pl.ds(v * L, L)] * wspl)
                return rc

            lax.fori_loop(0, EB // 8, _row, 0)

        pltpu.async_copy(src_hbm.at[s, 0], srcg.at[0], sem_i)
        pltpu.async_copy(dst_hbm.at[s, 0], dstg.at[0], sem_i)
        pltpu.make_async_copy(src_hbm.at[s, 0], srcg.at[0], sem_i).wait()
        pltpu.make_async_copy(dst_hbm.at[s, 0], dstg.at[0], sem_i).wait()
        _g_start(0, 0)
        _g_wait(0, 0)
        _idx_start(IG)
        _g_start(1, 1)
        _compute(0, 0)
        _s_start(0, 0)

        n_pairs = (nb - 2) // 2

        def _body(i, carry):
            for bpar in range(2):
                blk = 1 + 2 * i + bpar
                sl = (1 + bpar) % 2
                ot = 1 - sl
                _g_wait(blk, sl)
                _s_wait(blk - 1, ot)

                @pl.when((blk % IG == 0) & (blk < (ngrp - 1) * IG))
                def _():
                    _idx_start(blk + IG)

                @pl.when(((blk + 1) % IG == 0) & (blk + 1 < nb))
                def _():
                    _idx_wait(blk + 1)

                _g_start(blk + 1, ot)
                _compute(blk, sl)
                _s_start(blk, sl)
            return carry

        lax.fori_loop(0, n_pairs, _body, 0)

        for blk in range(1 + 2 * n_pairs, nb):
            sl = blk % 2
            ot = 1 - sl
            _g_wait(blk, sl)
            _s_wait(blk - 1, ot)
            if blk + 1 < nb:
                _g_start(blk + 1, ot)
            _compute(blk, sl)
            _s_start(blk, sl)
        _s_wait(nb - 1, (nb - 1) % 2)

    @pl.when(c == 0)
    def _():
        _run(NB0, G0, srcA_hbm, dstA_hbm)

    @pl.when(c == 1)
    def _():
        _run(NB1, G1, srcB_hbm, dstB_hbm)

    plsc.subcore_barrier()

    def _out(j, carry):
        pltpu.sync_copy(acc.at[pl.ds(base + j * 64, 64)],
                        out_hbm.at[c, pl.ds(base + j * 64, 64)])
        return carry

    lax.fori_loop(0, RPT // 64, _out, 0)


def _edge_pass(hext, adst, srcA, dstA, srcB, dstB):
    mesh = plsc.VectorSubcoreMesh(core_axis_name="c", subcore_axis_name="s")
    f = pl.kernel(
        _edge_body,
        out_type=jax.ShapeDtypeStruct((NC, NP, DE), jnp.float32),
        mesh=mesh,
        compiler_params=pltpu.CompilerParams(needs_layout_passes=False,
                                             use_tc_tiling_on_sc=False),
        scratch_types=[
            pltpu.VMEM((NP,), jnp.float32),
            pltpu.VMEM((2, IG, EB), jnp.int32),
            pltpu.VMEM((2, IG, EB), jnp.int32),
            pltpu.VMEM((2, EB, DE), jnp.float32),
            pltpu.VMEM((EB,), jnp.float32),
            pltpu.VMEM_SHARED((NP, DE), jnp.float32),
            pltpu.SemaphoreType.DMA,
            pltpu.SemaphoreType.DMA,
            pltpu.SemaphoreType.DMA,
        ],
    )
    return f(hext, adst, srcA, dstA, srcB, dstB)


def kernel(x, edge_index, batch, W1, att_src1, att_dst1, b1,
           W2, att_src2, att_dst2, b2, W3, att_src3, att_dst3, b3,
           fc_w, fc_b):
    f32 = jnp.float32
    x_pad = jnp.pad(x, ((0, NP - N), (0, 0)))
    loop = jnp.arange(N, dtype=jnp.int32)
    src = jnp.concatenate([edge_index[0].astype(jnp.int32), loop,
                           jnp.zeros((E_PAD - E_REAL,), jnp.int32)])
    dst = jnp.concatenate([edge_index[1].astype(jnp.int32), loop,
                           jnp.full((E_PAD - E_REAL,), N + 16, jnp.int32)])
    srcA = src[:E0].reshape(NS, G0, IG, EB)
    dstA = dst[:E0].reshape(NS, G0, IG, EB)
    srcB = src[E0:].reshape(NS, G1, IG, EB)
    dstB = dst[E0:].reshape(NS, G1, IG, EB)
    S = (batch[:, None] == jnp.arange(NG, dtype=batch.dtype)[None, :])
    S = jnp.pad(S.astype(f32), ((0, NP - N), (0, 0)))

    hext, asd = _dense_first(x_pad, W1, att_src1.reshape(1, D),
                             att_dst1.reshape(1, D))
    acc = _edge_pass(hext, asd[:, 0], srcA, dstA, srcB, dstB)
    hext, asd = _dense_mid(acc[0], acc[1], b1.reshape(1, D), W2,
                           att_src2.reshape(1, D), att_dst2.reshape(1, D))
    acc = _edge_pass(hext, asd[:, 0], srcA, dstA, srcB, dstB)
    hext, asd = _dense_mid(acc[0], acc[1], b2.reshape(1, D), W3,
                           att_src3.reshape(1, D), att_dst3.reshape(1, D))
    acc = _edge_pass(hext, asd[:, 0], srcA, dstA, srcB, dstB)
    return _final(acc[0], acc[1], b3.reshape(1, D), S, fc_w,
                  fc_b.reshape(1, NCLS))


# speedup vs baseline: 1.8698x; 1.8698x over previous
import jax
import jax.numpy as jnp
from jax import lax
from jax.experimental import pallas as pl
from jax.experimental.pallas import tpu as pltpu
from jax.experimental.pallas import tpu_sc as plsc

N = 10000
NP = 10240
D = 128
DE = 144
NG = 64
NCLS = 8
NC, NS, L = 2, 16, 16
NW = NC * NS
E_REAL = 320000 + N
EB = 64
IG = 27
G0 = 7
G1 = 5
NB0 = G0 * IG
NB1 = G1 * IG
E0 = NS * NB0 * EB
E_PAD = NS * (NB0 + NB1) * EB
RPT = NP // NS
BLK = 512
NBLK = NP // BLK


def _emit_outputs(h, ats_ref, atd_ref, hext_ref, asd_ref):
    hext_ref[:, :D] = h
    asrc = jnp.sum(h * ats_ref[...], axis=1, keepdims=True)
    adst = jnp.sum(h * atd_ref[...], axis=1, keepdims=True)
    col = lax.broadcasted_iota(jnp.int32, (h.shape[0], DE - D), 1)
    tail = jnp.where(col == 0, 1.0, jnp.where(col == 1, asrc, 0.0))
    hext_ref[:, D:] = tail
    col8 = lax.broadcasted_iota(jnp.int32, (h.shape[0], 8), 1)
    asd_ref[...] = jnp.where(col8 == 0, adst, 0.0)


def _dense_first_body(x_ref, w_ref, ats_ref, atd_ref, hext_ref, asd_ref):
    h = jnp.dot(x_ref[...], w_ref[...], preferred_element_type=jnp.float32)
    _emit_outputs(h, ats_ref, atd_ref, hext_ref, asd_ref)


def _node_activation(a_ref, b_ref, bias_ref):
    den = a_ref[:, D:D + 1] + b_ref[:, D:D + 1]
    num = a_ref[:, :D] + b_ref[:, :D]
    return jnp.maximum(num / jnp.maximum(den, 1e-16) + bias_ref[...], 0.0)


def _dense_mid_body(a_ref, b_ref, bias_ref, w_ref, ats_ref, atd_ref,
                    hext_ref, asd_ref):
    xv = _node_activation(a_ref, b_ref, bias_ref)
    h = jnp.dot(xv, w_ref[...], preferred_element_type=jnp.float32)
    _emit_outputs(h, ats_ref, atd_ref, hext_ref, asd_ref)


def _dense_outs():
    return (
        [pl.BlockSpec((BLK, DE), lambda i: (i, 0)),
         pl.BlockSpec((BLK, 8), lambda i: (i, 0))],
        [jax.ShapeDtypeStruct((NP, DE), jnp.float32),
         jax.ShapeDtypeStruct((NP, 8), jnp.float32)],
    )


def _dense_first(x_pad, W, ats, atd):
    out_specs, out_shape = _dense_outs()
    return pl.pallas_call(
        _dense_first_body,
        grid=(NBLK,),
        in_specs=[pl.BlockSpec((BLK, D), lambda i: (i, 0)),
                  pl.BlockSpec((D, D), lambda i: (0, 0)),
                  pl.BlockSpec((1, D), lambda i: (0, 0)),
                  pl.BlockSpec((1, D), lambda i: (0, 0))],
        out_specs=out_specs,
        out_shape=out_shape,
    )(x_pad, W, ats, atd)


def _dense_mid(accA, accB, bias, W, ats, atd):
    out_specs, out_shape = _dense_outs()
    return pl.pallas_call(
        _dense_mid_body,
        grid=(NBLK,),
        in_specs=[pl.BlockSpec((BLK, DE), lambda i: (i, 0)),
                  pl.BlockSpec((BLK, DE), lambda i: (i, 0)),
                  pl.BlockSpec((1, D), lambda i: (0, 0)),
                  pl.BlockSpec((D, D), lambda i: (0, 0)),
                  pl.BlockSpec((1, D), lambda i: (0, 0)),
                  pl.BlockSpec((1, D), lambda i: (0, 0))],
        out_specs=out_specs,
        out_shape=out_shape,
    )(accA, accB, bias, W, ats, atd)


def _final_body(a_ref, b_ref, bias_ref, s_ref, fcw_ref, fcb_ref, out_ref,
                pooled, cnt):
    i = pl.program_id(0)

    @pl.when(i == 0)
    def _():
        pooled[...] = jnp.zeros_like(pooled)
        cnt[...] = jnp.zeros_like(cnt)

    xv = _node_activation(a_ref, b_ref, bias_ref)
    sb = s_ref[...]
    pooled[...] += lax.dot_general(sb, xv, (((0,), (0,)), ((), ())),
                                   preferred_element_type=jnp.float32)
    cnt[...] += jnp.sum(sb, axis=0, keepdims=True)

    @pl.when(i == NBLK - 1)
    def _():
        c = jnp.maximum(cnt[...].reshape(NG, 1), 1.0)
        logits = jnp.dot(pooled[...] / c, fcw_ref[...],
                         preferred_element_type=jnp.float32) + fcb_ref[...]
        m = jnp.max(logits, axis=1, keepdims=True)
        e = jnp.exp(logits - m)
        out_ref[...] = e / jnp.sum(e, axis=1, keepdims=True)


def _final(accA, accB, bias, S, fcw, fcb):
    return pl.pallas_call(
        _final_body,
        grid=(NBLK,),
        in_specs=[pl.BlockSpec((BLK, DE), lambda i: (i, 0)),
                  pl.BlockSpec((BLK, DE), lambda i: (i, 0)),
                  pl.BlockSpec((1, D), lambda i: (0, 0)),
                  pl.BlockSpec((BLK, NG), lambda i: (i, 0)),
                  pl.BlockSpec((D, NCLS), lambda i: (0, 0)),
                  pl.BlockSpec((1, NCLS), lambda i: (0, 0))],
        out_specs=pl.BlockSpec((NG, NCLS), lambda i: (0, 0)),
        out_shape=jax.ShapeDtypeStruct((NG, NCLS), jnp.float32),
        scratch_shapes=[pltpu.VMEM((NG, D), jnp.float32),
                        pltpu.VMEM((1, NG), jnp.float32)],
    )(accA, accB, bias, S, fcw, fcb)


def _edge_body(hext_hbm, adst_hbm, srcA_hbm, dstA_hbm, srcB_hbm, dstB_hbm,
               out_hbm, adst_v, srcg, dstg, gbuf, wbuf, acc,
               sem_i, sem_g, sem_s):
    c = lax.axis_index("c")
    s = lax.axis_index("s")

    pltpu.async_copy(adst_hbm, adst_v, sem_i)
    zeros16 = jnp.zeros((L,), jnp.float32)

    def _zrow(r, carry):
        for v in range(DE // L):
            gbuf[0, r, pl.ds(v * L, L)] = zeros16
        return carry

    lax.fori_loop(0, EB, _zrow, 0)
    base = s * RPT

    def _zcopy(j, carry):
        pltpu.async_copy(gbuf.at[0], acc.at[pl.ds(base + j * EB, EB)], sem_s)
        return carry

    lax.fori_loop(0, RPT // EB, _zcopy, 0)
    pltpu.make_async_copy(adst_hbm, adst_v, sem_i).wait()

    def _zwait(j, carry):
        pltpu.make_async_copy(gbuf.at[0], acc.at[pl.ds(base + j * EB, EB)],
                              sem_s).wait()
        return carry

    lax.fori_loop(0, RPT // EB, _zwait, 0)
    plsc.subcore_barrier()

    def _run(nb, ngrp, src_hbm, dst_hbm):
        def _igrp(k):
            return (k // IG) % 2, k % IG

        def _idx_start(k):
            isl = (k // IG) % 2
            pltpu.async_copy(src_hbm.at[s, k // IG], srcg.at[isl], sem_i)
            pltpu.async_copy(dst_hbm.at[s, k // IG], dstg.at[isl], sem_i)

        def _idx_wait(k):
            isl = (k // IG) % 2
            pltpu.make_async_copy(src_hbm.at[s, k // IG], srcg.at[isl],
                                  sem_i).wait()
            pltpu.make_async_copy(dst_hbm.at[s, k // IG], dstg.at[isl],
                                  sem_i).wait()

        def _g_start(blk, sl):
            isl, bb = _igrp(blk)
            pltpu.async_copy(hext_hbm.at[srcg.at[isl, bb]], gbuf.at[sl],
                             sem_g)

        def _g_wait(blk, sl):
            isl, bb = _igrp(blk)
            pltpu.make_async_copy(hext_hbm.at[srcg.at[isl, bb]], gbuf.at[sl],
                                  sem_g).wait()

        def _s_start(blk, sl):
            isl, bb = _igrp(blk)
            pltpu.async_copy(gbuf.at[sl], acc.at[dstg.at[isl, bb]], sem_s,
                             add=True)

        def _s_wait(blk, sl):
            isl, bb = _igrp(blk)
            pltpu.make_async_copy(gbuf.at[sl], acc.at[dstg.at[isl, bb]],
                                  sem_s).wait()

        def _compute(blk, sl):
            isl, bb = _igrp(blk)
            for g in range(EB // L):
                rows = lax.iota(jnp.int32, L) + g * L
                cols = jnp.full((L,), D + 1, jnp.int32)
                asv = plsc.load_gather(gbuf.at[sl], [rows, cols])
                di = dstg[isl, bb, pl.ds(g * L, L)]
                adv = plsc.load_gather(adst_v, [di])
                al = asv + adv
                al = jnp.maximum(al, 0.0) + 0.2 * jnp.minimum(al, 0.0)
                wbuf[pl.ds(g * L, L)] = jnp.exp(jnp.minimum(al, 60.0))

            def _row(i, rc):
                for u in range(4):
                    r = 4 * i + u
                    wspl = plsc.load_gather(wbuf,
                                            [jnp.full((L,), r, jnp.int32)])
                    for v in range(DE // L):
                        gbuf[sl, r, pl.ds(v * L, L)] = (
                            gbuf[sl, r, pl.ds(v * L, L)] * wspl)
                return rc

            lax.fori_loop(0, EB // 4, _row, 0)

        pltpu.async_copy(src_hbm.at[s, 0], srcg.at[0], sem_i)
        pltpu.async_copy(dst_hbm.at[s, 0], dstg.at[0], sem_i)
        pltpu.make_async_copy(src_hbm.at[s, 0], srcg.at[0], sem_i).wait()
        pltpu.make_async_copy(dst_hbm.at[s, 0], dstg.at[0], sem_i).wait()
        _g_start(0, 0)
        _g_wait(0, 0)
        _idx_start(IG)
        _g_start(1, 1)
        _compute(0, 0)
        _s_start(0, 0)

        n_pairs = (nb - 2) // 2

        def _body(i, carry):
            for bpar in range(2):
                blk = 1 + 2 * i + bpar
                sl = (1 + bpar) % 2
                ot = 1 - sl
                _g_wait(blk, sl)
                _s_wait(blk - 1, ot)

                @pl.when((blk % IG == 0) & (blk < (ngrp - 1) * IG))
                def _():
                    _idx_start(blk + IG)

                @pl.when(((blk + 1) % IG == 0) & (blk + 1 < nb))
                def _():
                    _idx_wait(blk + 1)

                _g_start(blk + 1, ot)
                _compute(blk, sl)
                _s_start(blk, sl)
            return carry

        lax.fori_loop(0, n_pairs, _body, 0)

        for blk in range(1 + 2 * n_pairs, nb):
            sl = blk % 2
            ot = 1 - sl
            _g_wait(blk, sl)
            _s_wait(blk - 1, ot)
            if blk + 1 < nb:
                _g_start(blk + 1, ot)
            _compute(blk, sl)
            _s_start(blk, sl)
        _s_wait(nb - 1, (nb - 1) % 2)

    @pl.when(c == 0)
    def _():
        _run(NB0, G0, srcA_hbm, dstA_hbm)

    @pl.when(c == 1)
    def _():
        _run(NB1, G1, srcB_hbm, dstB_hbm)

    plsc.subcore_barrier()

    def _out(j, carry):
        pltpu.sync_copy(acc.at[pl.ds(base + j * 64, 64)],
                        out_hbm.at[c, pl.ds(base + j * 64, 64)])
        return carry

    lax.fori_loop(0, RPT // 64, _out, 0)


def _edge_pass(hext, adst, srcA, dstA, srcB, dstB):
    mesh = plsc.VectorSubcoreMesh(core_axis_name="c", subcore_axis_name="s")
    f = pl.kernel(
        _edge_body,
        out_type=jax.ShapeDtypeStruct((NC, NP, DE), jnp.float32),
        mesh=mesh,
        compiler_params=pltpu.CompilerParams(needs_layout_passes=False,
                                             use_tc_tiling_on_sc=False),
        scratch_types=[
            pltpu.VMEM((NP,), jnp.float32),
            pltpu.VMEM((2, IG, EB), jnp.int32),
            pltpu.VMEM((2, IG, EB), jnp.int32),
            pltpu.VMEM((2, EB, DE), jnp.float32),
            pltpu.VMEM((EB,), jnp.float32),
            pltpu.VMEM_SHARED((NP, DE), jnp.float32),
            pltpu.SemaphoreType.DMA,
            pltpu.SemaphoreType.DMA,
            pltpu.SemaphoreType.DMA,
        ],
    )
    return f(hext, adst, srcA, dstA, srcB, dstB)


def kernel(x, edge_index, batch, W1, att_src1, att_dst1, b1,
           W2, att_src2, att_dst2, b2, W3, att_src3, att_dst3, b3,
           fc_w, fc_b):
    f32 = jnp.float32
    x_pad = jnp.pad(x, ((0, NP - N), (0, 0)))
    loop = jnp.arange(N, dtype=jnp.int32)
    src = jnp.concatenate([edge_index[0].astype(jnp.int32), loop,
                           jnp.zeros((E_PAD - E_REAL,), jnp.int32)])
    dst = jnp.concatenate([edge_index[1].astype(jnp.int32), loop,
                           jnp.full((E_PAD - E_REAL,), N + 16, jnp.int32)])
    srcA = src[:E0].reshape(NS, G0, IG, EB)
    dstA = dst[:E0].reshape(NS, G0, IG, EB)
    srcB = src[E0:].reshape(NS, G1, IG, EB)
    dstB = dst[E0:].reshape(NS, G1, IG, EB)
    S = (batch[:, None] == jnp.arange(NG, dtype=batch.dtype)[None, :])
    S = jnp.pad(S.astype(f32), ((0, NP - N), (0, 0)))

    hext, asd = _dense_first(x_pad, W1, att_src1.reshape(1, D),
                             att_dst1.reshape(1, D))
    acc = _edge_pass(hext, asd[:, 0], srcA, dstA, srcB, dstB)
    hext, asd = _dense_mid(acc[0], acc[1], b1.reshape(1, D), W2,
                           att_src2.reshape(1, D), att_dst2.reshape(1, D))
    acc = _edge_pass(hext, asd[:, 0], srcA, dstA, srcB, dstB)
    hext, asd = _dense_mid(acc[0], acc[1], b2.reshape(1, D), W3,
                           att_src3.reshape(1, D), att_dst3.reshape(1, D))
    acc = _edge_pass(hext, asd[:, 0], srcA, dstA, srcB, dstB)
    return _final(acc[0], acc[1], b3.reshape(1, D), S, fc_w,
                  fc_b.reshape(1, NCLS))


# speedup vs baseline: 1.9350x; 1.0349x over previous
import jax
import jax.numpy as jnp
from jax import lax
from jax.experimental import pallas as pl
from jax.experimental.pallas import tpu as pltpu
from jax.experimental.pallas import tpu_sc as plsc

N = 10000
NP = 10240
D = 128
DE = 144
NG = 64
NCLS = 8
NC, NS, L = 2, 16, 16
NW = NC * NS
E_REAL = 320000 + N
EB = 72
IG = 18
G0 = 9
G1 = 7
NB0 = G0 * IG
NB1 = G1 * IG
E0 = NS * NB0 * EB
E_PAD = NS * (NB0 + NB1) * EB
RPT = NP // NS
BLK = 512
NBLK = NP // BLK


def _emit_outputs(h, ats_ref, atd_ref, hext_ref, asd_ref):
    hext_ref[:, :D] = h
    asrc = jnp.sum(h * ats_ref[...], axis=1, keepdims=True)
    adst = jnp.sum(h * atd_ref[...], axis=1, keepdims=True)
    col = lax.broadcasted_iota(jnp.int32, (h.shape[0], DE - D), 1)
    tail = jnp.where(col == 0, 1.0, jnp.where(col == 1, asrc, 0.0))
    hext_ref[:, D:] = tail
    col8 = lax.broadcasted_iota(jnp.int32, (h.shape[0], 8), 1)
    asd_ref[...] = jnp.where(col8 == 0, adst, 0.0)


def _dense_first_body(x_ref, w_ref, ats_ref, atd_ref, hext_ref, asd_ref):
    h = jnp.dot(x_ref[...], w_ref[...], preferred_element_type=jnp.float32)
    _emit_outputs(h, ats_ref, atd_ref, hext_ref, asd_ref)


def _node_activation(a_ref, b_ref, bias_ref):
    den = a_ref[:, D:D + 1] + b_ref[:, D:D + 1]
    num = a_ref[:, :D] + b_ref[:, :D]
    return jnp.maximum(num / jnp.maximum(den, 1e-16) + bias_ref[...], 0.0)


def _dense_mid_body(a_ref, b_ref, bias_ref, w_ref, ats_ref, atd_ref,
                    hext_ref, asd_ref):
    xv = _node_activation(a_ref, b_ref, bias_ref)
    h = jnp.dot(xv, w_ref[...], preferred_element_type=jnp.float32)
    _emit_outputs(h, ats_ref, atd_ref, hext_ref, asd_ref)


def _dense_outs():
    return (
        [pl.BlockSpec((BLK, DE), lambda i: (i, 0)),
         pl.BlockSpec((BLK, 8), lambda i: (i, 0))],
        [jax.ShapeDtypeStruct((NP, DE), jnp.float32),
         jax.ShapeDtypeStruct((NP, 8), jnp.float32)],
    )


def _dense_first(x_pad, W, ats, atd):
    out_specs, out_shape = _dense_outs()
    return pl.pallas_call(
        _dense_first_body,
        grid=(NBLK,),
        in_specs=[pl.BlockSpec((BLK, D), lambda i: (i, 0)),
                  pl.BlockSpec((D, D), lambda i: (0, 0)),
                  pl.BlockSpec((1, D), lambda i: (0, 0)),
                  pl.BlockSpec((1, D), lambda i: (0, 0))],
        out_specs=out_specs,
        out_shape=out_shape,
    )(x_pad, W, ats, atd)


def _dense_mid(accA, accB, bias, W, ats, atd):
    out_specs, out_shape = _dense_outs()
    return pl.pallas_call(
        _dense_mid_body,
        grid=(NBLK,),
        in_specs=[pl.BlockSpec((BLK, DE), lambda i: (i, 0)),
                  pl.BlockSpec((BLK, DE), lambda i: (i, 0)),
                  pl.BlockSpec((1, D), lambda i: (0, 0)),
                  pl.BlockSpec((D, D), lambda i: (0, 0)),
                  pl.BlockSpec((1, D), lambda i: (0, 0)),
                  pl.BlockSpec((1, D), lambda i: (0, 0))],
        out_specs=out_specs,
        out_shape=out_shape,
    )(accA, accB, bias, W, ats, atd)


def _final_body(a_ref, b_ref, bias_ref, s_ref, fcw_ref, fcb_ref, out_ref,
                pooled, cnt):
    i = pl.program_id(0)

    @pl.when(i == 0)
    def _():
        pooled[...] = jnp.zeros_like(pooled)
        cnt[...] = jnp.zeros_like(cnt)

    xv = _node_activation(a_ref, b_ref, bias_ref)
    sb = s_ref[...]
    pooled[...] += lax.dot_general(sb, xv, (((0,), (0,)), ((), ())),
                                   preferred_element_type=jnp.float32)
    cnt[...] += jnp.sum(sb, axis=0, keepdims=True)

    @pl.when(i == NBLK - 1)
    def _():
        c = jnp.maximum(cnt[...].reshape(NG, 1), 1.0)
        logits = jnp.dot(pooled[...] / c, fcw_ref[...],
                         preferred_element_type=jnp.float32) + fcb_ref[...]
        m = jnp.max(logits, axis=1, keepdims=True)
        e = jnp.exp(logits - m)
        out_ref[...] = e / jnp.sum(e, axis=1, keepdims=True)


def _final(accA, accB, bias, S, fcw, fcb):
    return pl.pallas_call(
        _final_body,
        grid=(NBLK,),
        in_specs=[pl.BlockSpec((BLK, DE), lambda i: (i, 0)),
                  pl.BlockSpec((BLK, DE), lambda i: (i, 0)),
                  pl.BlockSpec((1, D), lambda i: (0, 0)),
                  pl.BlockSpec((BLK, NG), lambda i: (i, 0)),
                  pl.BlockSpec((D, NCLS), lambda i: (0, 0)),
                  pl.BlockSpec((1, NCLS), lambda i: (0, 0))],
        out_specs=pl.BlockSpec((NG, NCLS), lambda i: (0, 0)),
        out_shape=jax.ShapeDtypeStruct((NG, NCLS), jnp.float32),
        scratch_shapes=[pltpu.VMEM((NG, D), jnp.float32),
                        pltpu.VMEM((1, NG), jnp.float32)],
    )(accA, accB, bias, S, fcw, fcb)


def _edge_body(hext_hbm, adst_hbm, srcA_hbm, dstA_hbm, srcB_hbm, dstB_hbm,
               out_hbm, adst_v, srcg, dstg, gbuf, wbuf, acc,
               sem_i, sem_g, sem_s):
    c = lax.axis_index("c")
    s = lax.axis_index("s")

    pltpu.async_copy(adst_hbm, adst_v, sem_i)
    zeros16 = jnp.zeros((L,), jnp.float32)

    def _zrow(r, carry):
        for v in range(DE // L):
            gbuf[0, r, pl.ds(v * L, L)] = zeros16
        return carry

    lax.fori_loop(0, EB, _zrow, 0)
    base = s * RPT

    def _zcopy(j, carry):
        pltpu.async_copy(gbuf.at[0, pl.ds(0, 64)],
                         acc.at[pl.ds(base + j * 64, 64)], sem_s)
        return carry

    lax.fori_loop(0, RPT // 64, _zcopy, 0)
    pltpu.make_async_copy(adst_hbm, adst_v, sem_i).wait()

    def _zwait(j, carry):
        pltpu.make_async_copy(gbuf.at[0, pl.ds(0, 64)],
                              acc.at[pl.ds(base + j * 64, 64)], sem_s).wait()
        return carry

    lax.fori_loop(0, RPT // 64, _zwait, 0)
    plsc.subcore_barrier()

    def _run(nb, ngrp, src_hbm, dst_hbm):
        def _igrp(k):
            return (k // IG) % 2, k % IG

        def _idx_start(k):
            isl = (k // IG) % 2
            pltpu.async_copy(src_hbm.at[s, k // IG], srcg.at[isl], sem_i)
            pltpu.async_copy(dst_hbm.at[s, k // IG], dstg.at[isl], sem_i)

        def _idx_wait(k):
            isl = (k // IG) % 2
            pltpu.make_async_copy(src_hbm.at[s, k // IG], srcg.at[isl],
                                  sem_i).wait()
            pltpu.make_async_copy(dst_hbm.at[s, k // IG], dstg.at[isl],
                                  sem_i).wait()

        def _g_start(blk, sl):
            isl, bb = _igrp(blk)
            pltpu.async_copy(hext_hbm.at[srcg.at[isl, bb]], gbuf.at[sl],
                             sem_g)

        def _g_wait(blk, sl):
            isl, bb = _igrp(blk)
            pltpu.make_async_copy(hext_hbm.at[srcg.at[isl, bb]], gbuf.at[sl],
                                  sem_g).wait()

        def _s_start(blk, sl):
            isl, bb = _igrp(blk)
            pltpu.async_copy(gbuf.at[sl], acc.at[dstg.at[isl, bb]], sem_s,
                             add=True)

        def _s_wait(blk, sl):
            isl, bb = _igrp(blk)
            pltpu.make_async_copy(gbuf.at[sl], acc.at[dstg.at[isl, bb]],
                                  sem_s).wait()

        def _compute(blk, sl):
            isl, bb = _igrp(blk)
            for g in range(EB // L):
                rows = lax.iota(jnp.int32, L) + g * L
                cols = jnp.full((L,), D + 1, jnp.int32)
                asv = plsc.load_gather(gbuf.at[sl], [rows, cols])
                di = dstg[isl, bb, pl.ds(g * L, L)]
                adv = plsc.load_gather(adst_v, [di])
                al = asv + adv
                al = jnp.maximum(al, 0.0) + 0.2 * jnp.minimum(al, 0.0)
                wbuf[pl.ds(g * L, L)] = jnp.exp(jnp.minimum(al, 60.0))

            def _row(i, rc):
                for u in range(4):
                    r = 4 * i + u
                    wspl = plsc.load_gather(wbuf,
                                            [jnp.full((L,), r, jnp.int32)])
                    for v in range(DE // L):
                        gbuf[sl, r, pl.ds(v * L, L)] = (
                            gbuf[sl, r, pl.ds(v * L, L)] * wspl)
                return rc

            lax.fori_loop(0, EB // 4, _row, 0)

        pltpu.async_copy(src_hbm.at[s, 0], srcg.at[0], sem_i)
        pltpu.async_copy(dst_hbm.at[s, 0], dstg.at[0], sem_i)
        pltpu.make_async_copy(src_hbm.at[s, 0], srcg.at[0], sem_i).wait()
        pltpu.make_async_copy(dst_hbm.at[s, 0], dstg.at[0], sem_i).wait()
        _g_start(0, 0)
        _g_wait(0, 0)
        _idx_start(IG)
        _g_start(1, 1)
        _compute(0, 0)
        _s_start(0, 0)

        n_pairs = (nb - 2) // 2

        def _body(i, carry):
            for bpar in range(2):
                blk = 1 + 2 * i + bpar
                sl = (1 + bpar) % 2
                ot = 1 - sl
                _g_wait(blk, sl)
                _s_wait(blk - 1, ot)

                @pl.when((blk % IG == 0) & (blk < (ngrp - 1) * IG))
                def _():
                    _idx_start(blk + IG)

                @pl.when(((blk + 1) % IG == 0) & (blk + 1 < nb))
                def _():
                    _idx_wait(blk + 1)

                _g_start(blk + 1, ot)
                _compute(blk, sl)
                _s_start(blk, sl)
            return carry

        lax.fori_loop(0, n_pairs, _body, 0)

        for blk in range(1 + 2 * n_pairs, nb):
            sl = blk % 2
            ot = 1 - sl
            _g_wait(blk, sl)
            _s_wait(blk - 1, ot)
            if blk + 1 < nb:
                _g_start(blk + 1, ot)
            _compute(blk, sl)
            _s_start(blk, sl)
        _s_wait(nb - 1, (nb - 1) % 2)

    @pl.when(c == 0)
    def _():
        _run(NB0, G0, srcA_hbm, dstA_hbm)

    @pl.when(c == 1)
    def _():
        _run(NB1, G1, srcB_hbm, dstB_hbm)

    plsc.subcore_barrier()

    def _out(j, carry):
        pltpu.sync_copy(acc.at[pl.ds(base + j * 64, 64)],
                        out_hbm.at[c, pl.ds(base + j * 64, 64)])
        return carry

    lax.fori_loop(0, RPT // 64, _out, 0)


def _edge_pass(hext, adst, srcA, dstA, srcB, dstB):
    mesh = plsc.VectorSubcoreMesh(core_axis_name="c", subcore_axis_name="s")
    f = pl.kernel(
        _edge_body,
        out_type=jax.ShapeDtypeStruct((NC, NP, DE), jnp.float32),
        mesh=mesh,
        compiler_params=pltpu.CompilerParams(needs_layout_passes=False,
                                             use_tc_tiling_on_sc=False),
        scratch_types=[
            pltpu.VMEM((NP,), jnp.float32),
            pltpu.VMEM((2, IG, EB), jnp.int32),
            pltpu.VMEM((2, IG, EB), jnp.int32),
            pltpu.VMEM((2, EB, DE), jnp.float32),
            pltpu.VMEM((EB,), jnp.float32),
            pltpu.VMEM_SHARED((NP, DE), jnp.float32),
            pltpu.SemaphoreType.DMA,
            pltpu.SemaphoreType.DMA,
            pltpu.SemaphoreType.DMA,
        ],
    )
    return f(hext, adst, srcA, dstA, srcB, dstB)


def kernel(x, edge_index, batch, W1, att_src1, att_dst1, b1,
           W2, att_src2, att_dst2, b2, W3, att_src3, att_dst3, b3,
           fc_w, fc_b):
    f32 = jnp.float32
    x_pad = jnp.pad(x, ((0, NP - N), (0, 0)))
    loop = jnp.arange(N, dtype=jnp.int32)
    src = jnp.concatenate([edge_index[0].astype(jnp.int32), loop,
                           jnp.zeros((E_PAD - E_REAL,), jnp.int32)])
    dst = jnp.concatenate([edge_index[1].astype(jnp.int32), loop,
                           jnp.full((E_PAD - E_REAL,), N + 16, jnp.int32)])
    srcA = src[:E0].reshape(NS, G0, IG, EB)
    dstA = dst[:E0].reshape(NS, G0, IG, EB)
    srcB = src[E0:].reshape(NS, G1, IG, EB)
    dstB = dst[E0:].reshape(NS, G1, IG, EB)
    S = (batch[:, None] == jnp.arange(NG, dtype=batch.dtype)[None, :])
    S = jnp.pad(S.astype(f32), ((0, NP - N), (0, 0)))

    hext, asd = _dense_first(x_pad, W1, att_src1.reshape(1, D),
                             att_dst1.reshape(1, D))
    acc = _edge_pass(hext, asd[:, 0], srcA, dstA, srcB, dstB)
    hext, asd = _dense_mid(acc[0], acc[1], b1.reshape(1, D), W2,
                           att_src2.reshape(1, D), att_dst2.reshape(1, D))
    acc = _edge_pass(hext, asd[:, 0], srcA, dstA, srcB, dstB)
    hext, asd = _dense_mid(acc[0], acc[1], b2.reshape(1, D), W3,
                           att_src3.reshape(1, D), att_dst3.reshape(1, D))
    acc = _edge_pass(hext, asd[:, 0], srcA, dstA, srcB, dstB)
    return _final(acc[0], acc[1], b3.reshape(1, D), S, fc_w,
                  fc_b.reshape(1, NCLS))
